# Initial kernel scaffold; baseline (speedup 1.0000x reference)
import jax
import jax.numpy as jnp
from jax.experimental import pallas as pl


def kernel(probs, bbox_deltas, img_info):
    raise NotImplementedError("write your pallas kernel here")


# speedup vs baseline: 16.7574x; 16.7574x over previous
import numpy as np
import jax
import jax.numpy as jnp
from jax.experimental import pallas as pl
from jax.experimental.pallas import tpu as pltpu

_FEATURE_STRIDES = [8]
_FEATURE_SHAPES = [(64, 64)]
_ANCHOR_SCALES = np.array([8.0, 16.0, 32.0], dtype=np.float32)
_ANCHOR_RATIOS = np.array([0.5, 1.0, 2.0], dtype=np.float32)
_PRE_NMS_TOP_N = 6000
_POST_NMS_TOP_N = 300
_NMS_THRESH = 0.7
_MIN_SIZE = 16.0
_LOG_MAX = float(np.log(1000.0 / 16.0))

_N = 36864
_R = 288
_C = 128


def _make_base_anchors(base_size, ratios, scales):
    w = h = float(base_size)
    x_ctr = y_ctr = (base_size - 1) * 0.5
    size = w * h
    size_ratios = size / ratios
    ws = np.round(np.sqrt(size_ratios))
    hs = np.round(ws * ratios)
    anchors = []
    for wi, hi in zip(ws, hs):
        for s in scales:
            wss = wi * s
            hss = hi * s
            anchors.append([x_ctr - 0.5 * (wss - 1), y_ctr - 0.5 * (hss - 1),
                            x_ctr + 0.5 * (wss - 1), y_ctr + 0.5 * (hss - 1)])
    return np.array(anchors, dtype=np.float32)


def _make_all_anchors():
    out = []
    for (fh, fw), stride in zip(_FEATURE_SHAPES, _FEATURE_STRIDES):
        base = _make_base_anchors(stride, _ANCHOR_RATIOS, _ANCHOR_SCALES)
        sx = np.arange(fw) * stride
        sy = np.arange(fh) * stride
        gx, gy = np.meshgrid(sx, sy)
        shifts = np.stack([gx.ravel(), gy.ravel(), gx.ravel(), gy.ravel()],
                          axis=1).astype(np.float32)
        out.append((shifts[:, None, :] + base[None, :, :]).reshape(-1, 4))
    return np.concatenate(out, axis=0)


_A = _make_all_anchors()
_WA = (_A[:, 2] - _A[:, 0] + np.float32(1.0)).reshape(_R, _C)
_HA = (_A[:, 3] - _A[:, 1] + np.float32(1.0)).reshape(_R, _C)
_CXA = (_A[:, 0] + np.float32(0.5) * _WA.reshape(-1)).reshape(_R, _C)
_CYA = (_A[:, 1] + np.float32(0.5) * _HA.reshape(-1)).reshape(_R, _C)


def _rpn_body(hw_ref, s_ref, dx_ref, dy_ref, dw_ref, dh_ref,
              wa_ref, ha_ref, cxa_ref, cya_ref, out_ref,
              x1s, y1s, x2s, y2s, ms_ref):
    h1 = hw_ref[0, 0]
    w1 = hw_ref[0, 1]
    wa = wa_ref[...]
    ha = ha_ref[...]
    s = s_ref[0]
    dx = dx_ref[0]
    dy = dy_ref[0]
    dw = jnp.minimum(dw_ref[0], _LOG_MAX)
    dh = jnp.minimum(dh_ref[0], _LOG_MAX)

    pcx = dx * wa + cxa_ref[...]
    pcy = dy * ha + cya_ref[...]
    pw = jnp.exp(dw) * wa
    ph = jnp.exp(dh) * ha
    x1 = jnp.clip(pcx - 0.5 * pw, 0.0, w1)
    y1 = jnp.clip(pcy - 0.5 * ph, 0.0, h1)
    x2 = jnp.clip(pcx + 0.5 * pw, 0.0, w1)
    y2 = jnp.clip(pcy + 0.5 * ph, 0.0, h1)
    ws = x2 - x1 + 1.0
    hs = y2 - y1 + 1.0
    s = jnp.where((ws >= _MIN_SIZE) & (hs >= _MIN_SIZE), s, -1.0)
    areas = ws * hs
    x1s[...] = x1
    y1s[...] = y1
    x2s[...] = x2
    y2s[...] = y2

    ki = jax.lax.bitcast_convert_type(s, jnp.int32)
    key = jnp.where(ki < 0, ki ^ jnp.int32(0x7FFFFFFF), ki)

    lin = (jax.lax.broadcasted_iota(jnp.int32, (_R, _C), 0) * _C
           + jax.lax.broadcasted_iota(jnp.int32, (_R, _C), 1))

    def bs1(_, lohi):
        lo, hi = lohi
        mid = (lo & hi) + ((lo ^ hi) >> 1)
        cnt = jnp.sum(jnp.where(key > mid, 1.0, 0.0))
        pred = cnt < _PRE_NMS_TOP_N
        return jnp.where(pred, lo, mid), jnp.where(pred, mid, hi)

    lo, hi = jax.lax.fori_loop(
        0, 32, bs1, (jnp.int32(-2**31), jnp.int32(2**31 - 1)))
    tau = hi
    cnt_gt = jnp.sum(jnp.where(key > tau, 1.0, 0.0))
    rem = jnp.float32(_PRE_NMS_TOP_N) - cnt_gt
    eqm = key == tau

    def bs2(_, lohi):
        lo, hi = lohi
        mid = (lo & hi) + ((lo ^ hi) >> 1)
        cnt = jnp.sum(jnp.where(eqm & (lin <= mid), 1.0, 0.0))
        pred = cnt >= rem
        return jnp.where(pred, lo, mid), jnp.where(pred, mid, hi)

    lo2, hi2 = jax.lax.fori_loop(
        0, 17, bs2, (jnp.int32(-1), jnp.int32(_N - 1)))
    istar = hi2

    cand = (key > tau) | (eqm & (lin <= istar))

    lane512 = jax.lax.broadcasted_iota(jnp.int32, (1, 512), 1)
    neg_inf = jnp.float32(-jnp.inf)
    ms_ref[...] = jnp.where(cand, s, neg_inf)

    def nms_body(i, carry):
        ax1, ay1, ax2, ay2 = carry
        masked = ms_ref[...]
        m = jnp.max(masked)
        has = m > neg_inf
        pick = jnp.where((masked == m) & (masked > neg_inf), lin, jnp.int32(_N))
        idx = jnp.minimum(jnp.min(pick), jnp.int32(_N - 1))
        rr = idx // _C
        cc = idx - rr * _C
        sel = jax.lax.broadcasted_iota(jnp.int32, (1, _C), 1) == cc
        bx1 = jnp.sum(jnp.where(sel, x1s[pl.ds(rr, 1), :], 0.0))
        by1 = jnp.sum(jnp.where(sel, y1s[pl.ds(rr, 1), :], 0.0))
        bx2 = jnp.sum(jnp.where(sel, x2s[pl.ds(rr, 1), :], 0.0))
        by2 = jnp.sum(jnp.where(sel, y2s[pl.ds(rr, 1), :], 0.0))
        barea = (bx2 - bx1 + 1.0) * (by2 - by1 + 1.0)
        xx1 = jnp.maximum(bx1, x1)
        yy1 = jnp.maximum(by1, y1)
        xx2 = jnp.minimum(bx2, x2)
        yy2 = jnp.minimum(by2, y2)
        iw = jnp.maximum(0.0, xx2 - xx1 + 1.0)
        ih = jnp.maximum(0.0, yy2 - yy1 + 1.0)
        inter = iw * ih
        iou = inter / (barea + areas - inter)
        ms_ref[...] = jnp.where((iou <= _NMS_THRESH) & has, masked, neg_inf)
        cm = lane512 == i
        zero = jnp.float32(0.0)
        ax1 = ax1 + jnp.where(cm, jnp.where(has, bx1, zero), zero)
        ay1 = ay1 + jnp.where(cm, jnp.where(has, by1, zero), zero)
        ax2 = ax2 + jnp.where(cm, jnp.where(has, bx2, zero), zero)
        ay2 = ay2 + jnp.where(cm, jnp.where(has, by2, zero), zero)
        return ax1, ay1, ax2, ay2

    z = jnp.zeros((1, 512), jnp.float32)
    ax1, ay1, ax2, ay2 = jax.lax.fori_loop(
        0, _POST_NMS_TOP_N, nms_body, (z, z, z, z))

    out_ref[0] = jnp.concatenate(
        [ax1, ay1, ax2, ay2, z, z, z, z], axis=0)


def kernel(probs, bbox_deltas, img_info):
    scores = probs[:, :, 1].reshape(2, _R, _C)
    dx = bbox_deltas[:, :, 0].reshape(2, _R, _C)
    dy = bbox_deltas[:, :, 1].reshape(2, _R, _C)
    dwv = bbox_deltas[:, :, 2].reshape(2, _R, _C)
    dhv = bbox_deltas[:, :, 3].reshape(2, _R, _C)
    hw = jnp.pad(img_info[0:1, 0:2] - 1.0, ((0, 7), (0, 126)))

    batch_spec = pl.BlockSpec((1, _R, _C), lambda b: (b, 0, 0))
    const_spec = pl.BlockSpec((_R, _C), lambda b: (0, 0))
    out = pl.pallas_call(
        _rpn_body,
        grid=(2,),
        in_specs=[
            pl.BlockSpec((8, 128), lambda b: (0, 0)),
            batch_spec, batch_spec, batch_spec, batch_spec, batch_spec,
            const_spec, const_spec, const_spec, const_spec,
        ],
        out_specs=pl.BlockSpec((1, 8, 512), lambda b: (b, 0, 0)),
        out_shape=jax.ShapeDtypeStruct((2, 8, 512), jnp.float32),
        scratch_shapes=[pltpu.VMEM((_R, _C), jnp.float32)] * 5,
    )(hw, scores, dx, dy, dwv, dhv,
      jnp.asarray(_WA), jnp.asarray(_HA), jnp.asarray(_CXA), jnp.asarray(_CYA))

    boxes = jnp.transpose(out[:, 0:4, 0:_POST_NMS_TOP_N], (0, 2, 1))
    bcol = jnp.broadcast_to(
        jnp.arange(2, dtype=jnp.float32)[:, None, None], (2, _POST_NMS_TOP_N, 1))
    return jnp.concatenate([bcol, boxes], axis=2)


# speedup vs baseline: 18.9502x; 1.1309x over previous
import numpy as np
import jax
import jax.numpy as jnp
from jax.experimental import pallas as pl
from jax.experimental.pallas import tpu as pltpu

_FEATURE_STRIDES = [8]
_FEATURE_SHAPES = [(64, 64)]
_ANCHOR_SCALES = np.array([8.0, 16.0, 32.0], dtype=np.float32)
_ANCHOR_RATIOS = np.array([0.5, 1.0, 2.0], dtype=np.float32)
_PRE_NMS_TOP_N = 6000
_POST_NMS_TOP_N = 300
_NMS_THRESH = 0.7
_MIN_SIZE = 16.0
_LOG_MAX = float(np.log(1000.0 / 16.0))

_N = 36864
_R = 288
_C = 128


def _make_base_anchors(base_size, ratios, scales):
    w = h = float(base_size)
    x_ctr = y_ctr = (base_size - 1) * 0.5
    size = w * h
    size_ratios = size / ratios
    ws = np.round(np.sqrt(size_ratios))
    hs = np.round(ws * ratios)
    anchors = []
    for wi, hi in zip(ws, hs):
        for s in scales:
            wss = wi * s
            hss = hi * s
            anchors.append([x_ctr - 0.5 * (wss - 1), y_ctr - 0.5 * (hss - 1),
                            x_ctr + 0.5 * (wss - 1), y_ctr + 0.5 * (hss - 1)])
    return np.array(anchors, dtype=np.float32)


def _make_all_anchors():
    out = []
    for (fh, fw), stride in zip(_FEATURE_SHAPES, _FEATURE_STRIDES):
        base = _make_base_anchors(stride, _ANCHOR_RATIOS, _ANCHOR_SCALES)
        sx = np.arange(fw) * stride
        sy = np.arange(fh) * stride
        gx, gy = np.meshgrid(sx, sy)
        shifts = np.stack([gx.ravel(), gy.ravel(), gx.ravel(), gy.ravel()],
                          axis=1).astype(np.float32)
        out.append((shifts[:, None, :] + base[None, :, :]).reshape(-1, 4))
    return np.concatenate(out, axis=0)


_A = _make_all_anchors()
_WA = (_A[:, 2] - _A[:, 0] + np.float32(1.0)).reshape(_R, _C)
_HA = (_A[:, 3] - _A[:, 1] + np.float32(1.0)).reshape(_R, _C)
_CXA = (_A[:, 0] + np.float32(0.5) * _WA.reshape(-1)).reshape(_R, _C)
_CYA = (_A[:, 1] + np.float32(0.5) * _HA.reshape(-1)).reshape(_R, _C)


def _rpn_body(hw_ref, s_ref, dx_ref, dy_ref, dw_ref, dh_ref,
              wa_ref, ha_ref, cxa_ref, cya_ref, out_ref,
              x1s0, y1s0, x2s0, y2s0, ms0,
              x1s1, y1s1, x2s1, y2s1, ms1):
    h1 = hw_ref[0, 0]
    w1 = hw_ref[0, 1]
    wa = wa_ref[...]
    ha = ha_ref[...]

    lin = (jax.lax.broadcasted_iota(jnp.int32, (_R, _C), 0) * _C
           + jax.lax.broadcasted_iota(jnp.int32, (_R, _C), 1))

    def decode(b):
        s = s_ref[b]
        dx = dx_ref[b]
        dy = dy_ref[b]
        dw = jnp.minimum(dw_ref[b], _LOG_MAX)
        dh = jnp.minimum(dh_ref[b], _LOG_MAX)
        pcx = dx * wa + cxa_ref[...]
        pcy = dy * ha + cya_ref[...]
        pw = jnp.exp(dw) * wa
        ph = jnp.exp(dh) * ha
        x1 = jnp.clip(pcx - 0.5 * pw, 0.0, w1)
        y1 = jnp.clip(pcy - 0.5 * ph, 0.0, h1)
        x2 = jnp.clip(pcx + 0.5 * pw, 0.0, w1)
        y2 = jnp.clip(pcy + 0.5 * ph, 0.0, h1)
        ws = x2 - x1 + 1.0
        hs = y2 - y1 + 1.0
        s = jnp.where((ws >= _MIN_SIZE) & (hs >= _MIN_SIZE), s, -1.0)
        areas = ws * hs
        ki = jax.lax.bitcast_convert_type(s, jnp.int32)
        key = jnp.where(ki < 0, ki ^ jnp.int32(0x7FFFFFFF), ki)
        return s, x1, y1, x2, y2, areas, key

    sA, x1A, y1A, x2A, y2A, arA, keyA = decode(0)
    sB, x1B, y1B, x2B, y2B, arB, keyB = decode(1)
    x1s0[...] = x1A
    y1s0[...] = y1A
    x2s0[...] = x2A
    y2s0[...] = y2A
    x1s1[...] = x1B
    y1s1[...] = y1B
    x2s1[...] = x2B
    y2s1[...] = y2B

    def bs1(_, st):
        loA, hiA, loB, hiB = st
        midA = (loA & hiA) + ((loA ^ hiA) >> 1)
        midB = (loB & hiB) + ((loB ^ hiB) >> 1)
        cntA = jnp.sum(jnp.where(keyA > midA, 1.0, 0.0))
        cntB = jnp.sum(jnp.where(keyB > midB, 1.0, 0.0))
        pA = cntA < _PRE_NMS_TOP_N
        pB = cntB < _PRE_NMS_TOP_N
        return (jnp.where(pA, loA, midA), jnp.where(pA, midA, hiA),
                jnp.where(pB, loB, midB), jnp.where(pB, midB, hiB))

    ilo = jnp.int32(-2**31)
    ihi = jnp.int32(2**31 - 1)
    _, tauA, _, tauB = jax.lax.fori_loop(0, 32, bs1, (ilo, ihi, ilo, ihi))
    remA = jnp.float32(_PRE_NMS_TOP_N) - jnp.sum(jnp.where(keyA > tauA, 1.0, 0.0))
    remB = jnp.float32(_PRE_NMS_TOP_N) - jnp.sum(jnp.where(keyB > tauB, 1.0, 0.0))
    eqA = keyA == tauA
    eqB = keyB == tauB

    def bs2(_, st):
        loA, hiA, loB, hiB = st
        midA = (loA & hiA) + ((loA ^ hiA) >> 1)
        midB = (loB & hiB) + ((loB ^ hiB) >> 1)
        cntA = jnp.sum(jnp.where(eqA & (lin <= midA), 1.0, 0.0))
        cntB = jnp.sum(jnp.where(eqB & (lin <= midB), 1.0, 0.0))
        pA = cntA >= remA
        pB = cntB >= remB
        return (jnp.where(pA, loA, midA), jnp.where(pA, midA, hiA),
                jnp.where(pB, loB, midB), jnp.where(pB, midB, hiB))

    m1 = jnp.int32(-1)
    nn = jnp.int32(_N - 1)
    _, isA, _, isB = jax.lax.fori_loop(0, 17, bs2, (m1, nn, m1, nn))

    neg_inf = jnp.float32(-jnp.inf)
    ms0[...] = jnp.where((keyA > tauA) | (eqA & (lin <= isA)), sA, neg_inf)
    ms1[...] = jnp.where((keyB > tauB) | (eqB & (lin <= isB)), sB, neg_inf)

    lane512 = jax.lax.broadcasted_iota(jnp.int32, (1, 512), 1)
    lane128 = jax.lax.broadcasted_iota(jnp.int32, (1, _C), 1)

    def step(ms_ref, x1s, y1s, x2s, y2s, x1, y1, x2, y2, areas, acc, i):
        ax1, ay1, ax2, ay2 = acc
        masked = ms_ref[...]
        m = jnp.max(masked)
        has = m > neg_inf
        pick = jnp.where((masked == m) & (masked > neg_inf), lin, jnp.int32(_N))
        idx = jnp.minimum(jnp.min(pick), jnp.int32(_N - 1))
        rr = idx // _C
        cc = idx - rr * _C
        sel = lane128 == cc
        bx1 = jnp.sum(jnp.where(sel, x1s[pl.ds(rr, 1), :], 0.0))
        by1 = jnp.sum(jnp.where(sel, y1s[pl.ds(rr, 1), :], 0.0))
        bx2 = jnp.sum(jnp.where(sel, x2s[pl.ds(rr, 1), :], 0.0))
        by2 = jnp.sum(jnp.where(sel, y2s[pl.ds(rr, 1), :], 0.0))
        barea = (bx2 - bx1 + 1.0) * (by2 - by1 + 1.0)
        xx1 = jnp.maximum(bx1, x1)
        yy1 = jnp.maximum(by1, y1)
        xx2 = jnp.minimum(bx2, x2)
        yy2 = jnp.minimum(by2, y2)
        iw = jnp.maximum(0.0, xx2 - xx1 + 1.0)
        ih = jnp.maximum(0.0, yy2 - yy1 + 1.0)
        inter = iw * ih
        iou = inter / (barea + areas - inter)
        ms_ref[...] = jnp.where((iou <= _NMS_THRESH) & has, masked, neg_inf)
        cm = lane512 == i
        zero = jnp.float32(0.0)
        ax1 = ax1 + jnp.where(cm, jnp.where(has, bx1, zero), zero)
        ay1 = ay1 + jnp.where(cm, jnp.where(has, by1, zero), zero)
        ax2 = ax2 + jnp.where(cm, jnp.where(has, bx2, zero), zero)
        ay2 = ay2 + jnp.where(cm, jnp.where(has, by2, zero), zero)
        return ax1, ay1, ax2, ay2

    def nms_body(i, carry):
        accA, accB = carry
        accA = step(ms0, x1s0, y1s0, x2s0, y2s0,
                    x1A, y1A, x2A, y2A, arA, accA, i)
        accB = step(ms1, x1s1, y1s1, x2s1, y2s1,
                    x1B, y1B, x2B, y2B, arB, accB, i)
        return accA, accB

    z = jnp.zeros((1, 512), jnp.float32)
    accA, accB = jax.lax.fori_loop(
        0, _POST_NMS_TOP_N, nms_body, ((z, z, z, z), (z, z, z, z)))

    out_ref[0] = jnp.concatenate(
        [accA[0], accA[1], accA[2], accA[3], z, z, z, z], axis=0)
    out_ref[1] = jnp.concatenate(
        [accB[0], accB[1], accB[2], accB[3], z, z, z, z], axis=0)


def kernel(probs, bbox_deltas, img_info):
    scores = probs[:, :, 1].reshape(2, _R, _C)
    dx = bbox_deltas[:, :, 0].reshape(2, _R, _C)
    dy = bbox_deltas[:, :, 1].reshape(2, _R, _C)
    dwv = bbox_deltas[:, :, 2].reshape(2, _R, _C)
    dhv = bbox_deltas[:, :, 3].reshape(2, _R, _C)
    hw = jnp.pad(img_info[0:1, 0:2] - 1.0, ((0, 7), (0, 126)))

    full_spec = pl.BlockSpec((2, _R, _C), lambda: (0, 0, 0))
    const_spec = pl.BlockSpec((_R, _C), lambda: (0, 0))
    out = pl.pallas_call(
        _rpn_body,
        in_specs=[
            pl.BlockSpec((8, 128), lambda: (0, 0)),
            full_spec, full_spec, full_spec, full_spec, full_spec,
            const_spec, const_spec, const_spec, const_spec,
        ],
        out_specs=pl.BlockSpec((2, 8, 512), lambda: (0, 0, 0)),
        out_shape=jax.ShapeDtypeStruct((2, 8, 512), jnp.float32),
        scratch_shapes=[pltpu.VMEM((_R, _C), jnp.float32)] * 10,
    )(hw, scores, dx, dy, dwv, dhv,
      jnp.asarray(_WA), jnp.asarray(_HA), jnp.asarray(_CXA), jnp.asarray(_CYA))

    boxes = jnp.transpose(out[:, 0:4, 0:_POST_NMS_TOP_N], (0, 2, 1))
    bcol = jnp.broadcast_to(
        jnp.arange(2, dtype=jnp.float32)[:, None, None], (2, _POST_NMS_TOP_N, 1))
    return jnp.concatenate([bcol, boxes], axis=2)
